# baseline (speedup 1.0000x reference)
import functools

import jax
import jax.numpy as jnp
from jax.experimental import pallas as pl
from jax.experimental.pallas import tpu as pltpu


def _mp_body(h_ref, a_ref, e_ref, wcat_ref, we_ref, b_ref, o_ref, *, d):
    h = h_ref[0]
    a = a_ref[0]
    e = e_ref[0]

    hw = jnp.dot(h, wcat_ref[...], preferred_element_type=jnp.float32)
    deg = jnp.sum(a, axis=1, keepdims=True)
    m = jnp.sum(a[:, :, None] * e, axis=1)
    me = jnp.dot(m, we_ref[...], preferred_element_type=jnp.float32)
    o_ref[0] = hw[:, :d] + deg * hw[:, d:] + me + b_ref[...]


def kernel(H, A, E, N, W_self, W_nei, bias):
    B, Nn, D = H.shape
    De = E.shape[-1]
    W_cat = jnp.concatenate([W_self, W_nei[:D]], axis=1)
    W_e = W_nei[D:]
    bias2 = bias[None, :]

    grid = (B,)
    out = pl.pallas_call(
        functools.partial(_mp_body, d=D),
        grid=grid,
        in_specs=[
            pl.BlockSpec((1, Nn, D), lambda a: (a, 0, 0)),
            pl.BlockSpec((1, Nn, Nn), lambda a: (a, 0, 0)),
            pl.BlockSpec((1, Nn, Nn, De), lambda a: (a, 0, 0, 0)),
            pl.BlockSpec((D, 2 * D), lambda a: (0, 0)),
            pl.BlockSpec((De, D), lambda a: (0, 0)),
            pl.BlockSpec((1, D), lambda a: (0, 0)),
        ],
        out_specs=pl.BlockSpec((1, Nn, D), lambda a: (a, 0, 0)),
        out_shape=jax.ShapeDtypeStruct((B, Nn, D), jnp.float32),
        compiler_params=pltpu.CompilerParams(
            dimension_semantics=("arbitrary",),
        ),
    )(H, A, E, W_cat, W_e, bias2)
    return out


# speedup vs baseline: 6.2296x; 6.2296x over previous
import functools

import jax
import jax.numpy as jnp
from jax.experimental import pallas as pl
from jax.experimental.pallas import tpu as pltpu


def _mp_body(h_ref, a_ref, et_ref, wcat_ref, we_ref, b_ref, o_ref, *, d):
    h = h_ref[0]
    a = a_ref[0]
    et = et_ref[0]

    hw = jnp.dot(h, wcat_ref[...], preferred_element_type=jnp.float32)
    deg = jnp.sum(a, axis=1, keepdims=True)
    m = jnp.sum(a[:, None, :] * et, axis=2)
    me = jnp.dot(m, we_ref[...], preferred_element_type=jnp.float32)
    o_ref[0] = hw[:, :d] + deg * hw[:, d:] + me + b_ref[...]


def kernel(H, A, E, N, W_self, W_nei, bias):
    B, Nn, D = H.shape
    De = E.shape[-1]
    Et = jnp.swapaxes(E, 2, 3)
    W_cat = jnp.concatenate([W_self, W_nei[:D]], axis=1)
    W_e = W_nei[D:]
    bias2 = bias[None, :]

    grid = (B,)
    out = pl.pallas_call(
        functools.partial(_mp_body, d=D),
        grid=grid,
        in_specs=[
            pl.BlockSpec((1, Nn, D), lambda a: (a, 0, 0)),
            pl.BlockSpec((1, Nn, Nn), lambda a: (a, 0, 0)),
            pl.BlockSpec((1, Nn, De, Nn), lambda a: (a, 0, 0, 0)),
            pl.BlockSpec((D, 2 * D), lambda a: (0, 0)),
            pl.BlockSpec((De, D), lambda a: (0, 0)),
            pl.BlockSpec((1, D), lambda a: (0, 0)),
        ],
        out_specs=pl.BlockSpec((1, Nn, D), lambda a: (a, 0, 0)),
        out_shape=jax.ShapeDtypeStruct((B, Nn, D), jnp.float32),
        compiler_params=pltpu.CompilerParams(
            dimension_semantics=("arbitrary",),
        ),
    )(H, A, Et, W_cat, W_e, bias2)
    return out
